# baseline (speedup 1.0000x reference)
import functools

import jax
import jax.numpy as jnp
from jax import lax
from jax.experimental import pallas as pl
from jax.experimental.pallas import tpu as pltpu
from jax.experimental.pallas import tpu_sc as plsc

_B, _C, _H, _W = 64, 96, 64, 64
_BT = 48
_SCB = _B - _BT
_TILE = _H * _W
_NW = 32
_NCH_TOT = _SCB * _C // _NW
_CCH = 4
_NCH = _NCH_TOT // _CCH
_LN2 = 0.6931471805599453
_BB = 2


def _tc_body(lab_ref, hp_ref, hprot_ref, out_ref):
    step = pl.program_id(0)
    iota = lax.broadcasted_iota(jnp.int32, (_C, _H, _W), 2)
    rev = (_W - 1) - iota

    for i in range(_BB):
        x = hp_ref[i]
        y = hprot_ref[i]
        r = lab_ref[step * _BB + i]

        idx1 = jnp.where(r == 0, iota, rev)
        idxy = jnp.where(r == 2, rev, iota)
        xt = jnp.swapaxes(x, 1, 2)
        a = jnp.where((r == 1) | (r == 2), xt, x)
        bb = jnp.take_along_axis(a, idx1, axis=2)
        c = jnp.swapaxes(bb, 1, 2)
        xr = jnp.where(r <= 1, bb, c)
        yg = jnp.take_along_axis(y, idxy, axis=2)

        diff = xr - yg
        out_ref[0, i, 0] = jnp.sum(diff * diff)
        out_ref[0, i, 1] = jnp.sum(xr * jnp.log(xr / jnp.maximum(yg, 1e-9)))


def _tc_call(hp, hp_rot, labs):
    grid_spec = pltpu.PrefetchScalarGridSpec(
        num_scalar_prefetch=1,
        grid=(_BT // _BB,),
        in_specs=[
            pl.BlockSpec((_BB, _C, _H, _W), lambda b, lab: (b, 0, 0, 0)),
            pl.BlockSpec((_BB, _C, _H, _W), lambda b, lab: (b, 0, 0, 0)),
        ],
        out_specs=[
            pl.BlockSpec(memory_space=pltpu.SMEM, block_shape=(1, _BB, 2),
                         index_map=lambda b, lab: (b, 0, 0)),
        ],
    )
    return pl.pallas_call(
        _tc_body,
        grid_spec=grid_spec,
        out_shape=[jax.ShapeDtypeStruct((_BT // _BB, _BB, 2), jnp.float32)],
    )(labs, hp, hp_rot)[0]


def _tile_pair_loss(xref, yref, rvec, l2v, klv):
    lane = lax.broadcasted_iota(jnp.int32, (16,), 0)
    is_odd = (rvec & 1) == 1
    is_hi = rvec >= 2

    def h_body(h, carry):
        l2c, klc = carry
        for wc in range(4):
            w = wc * 16 + lane
            i0 = h * 64 + w
            i1 = (63 * 64) + h - (w * 64)
            idx01 = jnp.where(is_odd, i1, i0)
            idx = jnp.where(is_hi, 4095 - idx01, idx01)
            xv = plsc.load_gather(xref, [idx])
            yv = yref[pl.ds(h * 64 + wc * 16, 16)]
            d = xv - yv
            l2c = l2c + d * d
            t = xv / jnp.maximum(yv, 1e-9)
            bits = plsc.bitcast(t, jnp.int32)
            e = (bits >> 23) - 127
            m = plsc.bitcast((bits & 0x007FFFFF) | 0x3F800000, jnp.float32)
            s = (m - 1.0) / (m + 1.0)
            s2 = s * s
            lg = s * (2.0 + s2 * (0.66666667 + s2 * (0.4 + s2 * (0.28571429
                      + s2 * 0.22222222))))
            klc = klc + xv * (lg + e.astype(jnp.float32) * _LN2)
        return l2c, klc

    return lax.fori_loop(0, _H, h_body, (l2v, klv))


def _sc_call(hp2, hprot2, labw):
    mesh = plsc.VectorSubcoreMesh(core_axis_name="c", subcore_axis_name="s")

    @functools.partial(
        pl.kernel,
        mesh=mesh,
        compiler_params=pltpu.CompilerParams(
            needs_layout_passes=False, use_tc_tiling_on_sc=False),
        out_type=[
            jax.ShapeDtypeStruct((_NW, 16), jnp.float32),
            jax.ShapeDtypeStruct((_NW, 16), jnp.float32),
        ],
        scratch_types=[
            pltpu.VMEM((2, _CCH, _TILE), jnp.float32),
            pltpu.VMEM((2, _CCH, _TILE), jnp.float32),
            pltpu.VMEM((_NCH, 16), jnp.int32),
            pltpu.SemaphoreType.DMA,
            pltpu.SemaphoreType.DMA,
        ],
    )
    def sc_kernel(hp_hbm, hprot_hbm, lab_hbm, l2_hbm, kl_hbm,
                  xb, yb, labv, semx, semy):
        wid = lax.axis_index("s") * 2 + lax.axis_index("c")
        pltpu.sync_copy(lab_hbm.at[pl.ds(wid * _NCH, _NCH)], labv)

        base_row = wid * _NCH_TOT

        def issue(ch, slot):
            row0 = base_row + ch * _CCH
            pltpu.async_copy(hp_hbm.at[pl.ds(row0, _CCH)], xb.at[slot], semx)
            pltpu.async_copy(hprot_hbm.at[pl.ds(row0, _CCH)], yb.at[slot],
                             semy)

        def drain(slot):
            pltpu.make_async_copy(
                hp_hbm.at[pl.ds(0, _CCH)], xb.at[slot], semx).wait()
            pltpu.make_async_copy(
                hprot_hbm.at[pl.ds(0, _CCH)], yb.at[slot], semy).wait()

        issue(0, 0)
        issue(1, 1)
        zero = jnp.zeros((16,), jnp.float32)

        def pair_body(i, carry):
            l2v, klv = carry
            for k in (0, 1):
                g = 2 * i + k
                drain(k)
                rvec = labv[g]
                for ci in range(_CCH):
                    l2v, klv = _tile_pair_loss(
                        xb.at[k, ci], yb.at[k, ci], rvec, l2v, klv)

                @pl.when(g + 2 < _NCH)
                def _():
                    issue(g + 2, k)
            return l2v, klv

        l2v, klv = lax.fori_loop(0, _NCH // 2, pair_body, (zero, zero))
        xb[0, 0, pl.ds(0, 16)] = l2v
        xb[0, 0, pl.ds(16, 16)] = klv
        pltpu.sync_copy(xb.at[0, 0, pl.ds(0, 16)], l2_hbm.at[wid])
        pltpu.sync_copy(xb.at[0, 0, pl.ds(16, 16)], kl_hbm.at[wid])

    return sc_kernel(hp2, hprot2, labw)


def kernel(hp, hp_rot, label_rot):
    labs = label_rot.astype(jnp.int32)
    tc_out = _tc_call(hp[:_BT], hp_rot[:_BT], labs[:_BT])

    hp2 = hp[_BT:].reshape(_SCB * _C, _TILE)
    hprot2 = hp_rot[_BT:].reshape(_SCB * _C, _TILE)
    chunk_batch = (jnp.arange(_NW * _NCH, dtype=jnp.int32)
                   * _CCH) // _C
    labw = jnp.broadcast_to(
        labs[_BT:][chunk_batch][:, None], (_NW * _NCH, 16))
    l2p, klp = _sc_call(hp2, hprot2, labw)

    l2 = tc_out[:, :, 0].sum() + l2p.sum()
    kl = tc_out[:, :, 1].sum() + klp.sum()
    return (kl / _B) * 0.4 + (l2 / (_B * _C * _H * _W)) * 0.6


# speedup vs baseline: 1.3682x; 1.3682x over previous
import jax
import jax.numpy as jnp
from jax import lax
from jax.experimental import pallas as pl
from jax.experimental.pallas import tpu as pltpu

_B, _C, _H, _W = 64, 96, 64, 64


def _body(lab_ref, hp_ref, hprot_ref, l2_ref, kl_ref):
    b = pl.program_id(0)
    x = hp_ref[0]
    y = hprot_ref[0]
    r = lab_ref[b]

    rev = jnp.broadcast_to(
        (_W - 1) - lax.broadcasted_iota(jnp.int32, (_C, _H, _W), 2),
        (_C, _H, _W))

    def _g(v):
        return jnp.take_along_axis(v, rev, axis=2)

    def _t(v):
        return jnp.swapaxes(v, 1, 2)

    def _sums(xr):
        diff = xr - y
        l2 = jnp.sum(diff * diff)
        kl = jnp.sum(xr * jnp.log(xr / jnp.maximum(y, 1e-9)))
        return l2, kl

    l2, kl = lax.switch(
        r,
        (
            lambda: _sums(x),
            lambda: _sums(_g(_t(x))),
            lambda: _sums(_g(_t(_g(_t(x))))),
            lambda: _sums(_t(_g(x))),
        ),
    )

    @pl.when(b == 0)
    def _():
        l2_ref[0, 0] = 0.0
        kl_ref[0, 0] = 0.0

    l2_ref[0, 0] += l2
    kl_ref[0, 0] += kl


def kernel(hp, hp_rot, label_rot):
    grid_spec = pltpu.PrefetchScalarGridSpec(
        num_scalar_prefetch=1,
        grid=(_B,),
        in_specs=[
            pl.BlockSpec((1, _C, _H, _W), lambda b, lab: (b, 0, 0, 0)),
            pl.BlockSpec((1, _C, _H, _W), lambda b, lab: (b, 0, 0, 0)),
        ],
        out_specs=[
            pl.BlockSpec(memory_space=pltpu.SMEM, block_shape=(1, 1),
                         index_map=lambda b, lab: (0, 0)),
            pl.BlockSpec(memory_space=pltpu.SMEM, block_shape=(1, 1),
                         index_map=lambda b, lab: (0, 0)),
        ],
    )
    l2, kl = pl.pallas_call(
        _body,
        grid_spec=grid_spec,
        out_shape=[
            jax.ShapeDtypeStruct((1, 1), jnp.float32),
            jax.ShapeDtypeStruct((1, 1), jnp.float32),
        ],
    )(label_rot.astype(jnp.int32), hp, hp_rot)
    kl_s = kl[0, 0] / _B
    l2_s = l2[0, 0] / (_B * _C * _H * _W)
    return kl_s * 0.4 + l2_s * 0.6
